# Initial kernel scaffold; baseline (speedup 1.0000x reference)
import jax
import jax.numpy as jnp
from jax.experimental import pallas as pl


def kernel(x, edge_index, edge_feats, W_in, b_in, g0, be0, Wq, bq, Wk, bk, Wv, bv, Wo, bo, We, gl, bl, W_out, b_out):
    raise NotImplementedError("write your pallas kernel here")


# speedup vs baseline: 23.4027x; 23.4027x over previous
import jax
import jax.numpy as jnp
from jax import lax
from jax.experimental import pallas as pl
from jax.experimental.pallas import tpu as pltpu
from jax.experimental.pallas import tpu_sc as plsc

N = 10000
E = 320000
D = 128
H = 8
HD = 16
L = 2
EDIM = 16
NP = 10240
EP = 327680
B = 128
CHUNKS = EP // B
NW = 32
CPT = CHUNKS // NW
NG = NP // 8
RB = 1024
EB = 2048
NPT = NP // 16
NGT = NG // 16


def _f32(x):
    return x.astype(jnp.float32)


def _dot(a, b):
    return jnp.dot(a, b, preferred_element_type=jnp.float32)


def _head_sum_sel():
    c = lax.broadcasted_iota(jnp.int32, (D, H), 0)
    h = lax.broadcasted_iota(jnp.int32, (D, H), 1)
    return _f32(c // HD == h)


def _rep_sel():
    h = lax.broadcasted_iota(jnp.int32, (H, D), 0)
    c = lax.broadcasted_iota(jnp.int32, (H, D), 1)
    return _f32(c // HD == h)


def _grp_sel():
    g = lax.broadcasted_iota(jnp.int32, (8, D), 0)
    c = lax.broadcasted_iota(jnp.int32, (8, D), 1)
    return _f32(c // 16 == g)


def _tile16_sel():
    j = lax.broadcasted_iota(jnp.int32, (16, D), 0)
    c = lax.broadcasted_iota(jnp.int32, (16, D), 1)
    return _f32(c % 16 == j)


def _pre_body(x_ref, w0_ref, w1t_ref, b_ref, g_ref, be_ref, o_ref):
    x = x_ref[...]
    nrm = jnp.sqrt(jnp.clip(jnp.sum(x * x, axis=-1, keepdims=True), 1e-12, None))
    en = jnp.exp(nrm)
    inv_en = 1.0 / en
    zt = 0.5 * (en + inv_en)
    zs = x * (0.5 * (en - inv_en) / nrm)
    s1 = zt * w0_ref[...] + _dot(zs, w1t_ref[...]) + b_ref[...]
    m = jnp.mean(s1, axis=-1, keepdims=True)
    v = jnp.mean((s1 - m) ** 2, axis=-1, keepdims=True)
    s2 = (s1 - m) / jnp.sqrt(v + 1e-5) * g_ref[...] + be_ref[...]
    o_ref[...] = jnp.maximum(s2, 0.0)


def _tc_pre(xp, w0, w1t, b, g, be):
    row = pl.BlockSpec((RB, D), lambda i: (i, 0))
    wfull = pl.BlockSpec((1, D), lambda i: (0, 0))
    wmat = pl.BlockSpec((D, D), lambda i: (0, 0))
    return pl.pallas_call(
        _pre_body,
        grid=(NP // RB,),
        in_specs=[row, wfull, wmat, wfull, wfull, wfull],
        out_specs=row,
        out_shape=jax.ShapeDtypeStruct((NP, D), jnp.float32),
    )(xp, w0, w1t, b, g, be)


def _tabs_body(h_ref, q0, q1t, qb, k0, k1t, kb, v0, v1t, vb,
               qo_ref, ko_ref, vo_ref):
    hs = h_ref[...]
    t = jnp.sqrt(jnp.sum(hs * hs, axis=-1, keepdims=True) + 1.0)
    qo_ref[...] = t * q0[...] + _dot(hs, q1t[...]) + qb[...]
    ko_ref[...] = 0.5 * (t * k0[...] + _dot(hs, k1t[...]) + kb[...])
    vo_ref[...] = t * v0[...] + _dot(hs, v1t[...]) + vb[...]


def _tc_tabs(hs, q0, q1t, qb, k0, k1t, kb, v0, v1t, vb):
    row = pl.BlockSpec((RB, D), lambda i: (i, 0))
    wfull = pl.BlockSpec((1, D), lambda i: (0, 0))
    wmat = pl.BlockSpec((D, D), lambda i: (0, 0))
    osh = jax.ShapeDtypeStruct((NP, D), jnp.float32)
    return pl.pallas_call(
        _tabs_body,
        grid=(NP // RB,),
        in_specs=[row] + [wfull, wmat, wfull] * 3,
        out_specs=[row, row, row],
        out_shape=[osh, osh, osh],
    )(hs, q0, q1t, qb, k0, k1t, kb, v0, v1t, vb)


def _gather_body(qtab, ktab, vtab, src2, dst2, qe, ke, ve,
                 idxs, idxd, qbuf, kbuf, vbuf, sq, sk, sv):
    cid = lax.axis_index("c")
    sid = lax.axis_index("s")
    w = sid * 2 + cid
    base = w * CPT
    pltpu.sync_copy(dst2.at[pl.ds(base, CPT)], idxd)
    pltpu.sync_copy(src2.at[pl.ds(base, CPT)], idxs)

    def step(c, carry):
        cq = pltpu.async_copy(qtab.at[idxd.at[c]], qbuf, sq)
        ck = pltpu.async_copy(ktab.at[idxs.at[c]], kbuf, sk)
        cv = pltpu.async_copy(vtab.at[idxs.at[c]], vbuf, sv)
        cq.wait()
        ck.wait()
        cv.wait()
        pltpu.sync_copy(qbuf, qe.at[base + c])
        pltpu.sync_copy(kbuf, ke.at[base + c])
        pltpu.sync_copy(vbuf, ve.at[base + c])
        return carry

    lax.fori_loop(0, CPT, step, 0)


def _sc_gather(qtab, ktab, vtab, src2, dst2):
    mesh = plsc.VectorSubcoreMesh(core_axis_name="c", subcore_axis_name="s")
    osh = jax.ShapeDtypeStruct((CHUNKS, B, D), jnp.float32)
    f = pl.kernel(
        _gather_body,
        out_type=[osh, osh, osh],
        mesh=mesh,
        scratch_types=[
            pltpu.VMEM((CPT, B), jnp.int32),
            pltpu.VMEM((CPT, B), jnp.int32),
            pltpu.VMEM((B, D), jnp.float32),
            pltpu.VMEM((B, D), jnp.float32),
            pltpu.VMEM((B, D), jnp.float32),
            pltpu.SemaphoreType.DMA,
            pltpu.SemaphoreType.DMA,
            pltpu.SemaphoreType.DMA,
        ],
    )
    return f(qtab, ktab, vtab, src2, dst2)


def _score_body(qe_ref, ke_ref, ve_ref, ef_ref, mask_ref, wet_ref,
                p1_ref, p2_ref):
    qe = qe_ref[...]
    ke = ke_ref[...]
    ve = ve_ref[...]
    ssp = _head_sum_sel()
    tq = jnp.sqrt(_dot(qe * qe, ssp) + 1.0)
    tk = jnp.sqrt(4.0 * _dot(ke * ke, ssp) + 1.0)
    sc = _dot(qe * ke, ssp) - 0.5 * tq * tk
    efs = _dot(ef_ref[...], wet_ref[...]) + 0.5
    ex = jnp.exp(sc + efs)
    rep = _rep_sel()
    p1_ref[...] = _dot(ex, rep) * ve
    tv = jnp.sqrt(_dot(ve * ve, ssp) + 1.0)
    ext = jnp.concatenate([ex, ex * tv], axis=-1)
    mrep = _dot(mask_ref[...], _grp_sel())
    erep = _dot(ext, _tile16_sel())
    p2_ref[...] = mrep * erep


def _tc_score(qe, ke, ve, efp, mask8, wet):
    erow = pl.BlockSpec((EB, D), lambda i: (i, 0))
    efrow = pl.BlockSpec((EB, EDIM), lambda i: (i, 0))
    mrow = pl.BlockSpec((EB, 8), lambda i: (i, 0))
    wspec = pl.BlockSpec((EDIM, H), lambda i: (0, 0))
    osh = jax.ShapeDtypeStruct((EP, D), jnp.float32)
    return pl.pallas_call(
        _score_body,
        grid=(EP // EB,),
        in_specs=[erow, erow, erow, efrow, mrow, wspec],
        out_specs=[erow, erow],
        out_shape=[osh, osh],
    )(qe, ke, ve, efp, mask8, wet)


def _make_scatter(nrows):
    nrt = nrows // 16

    def body(p3, idx2, zsrc, acc_out, idxv, pbuf, acc_sh):
        cid = lax.axis_index("c")
        sid = lax.axis_index("s")
        w = sid * 2 + cid
        base = w * CPT
        pltpu.sync_copy(idx2.at[pl.ds(base, CPT)], idxv)
        r0 = sid * nrt
        pltpu.sync_copy(zsrc.at[pl.ds(0, nrt)], acc_sh.at[pl.ds(r0, nrt)])
        plsc.subcore_barrier()

        def step(c, carry):
            pltpu.sync_copy(p3.at[base + c], pbuf)
            pltpu.sync_copy(pbuf, acc_sh.at[idxv.at[c]], add=True)
            return carry

        lax.fori_loop(0, CPT, step, 0)
        plsc.subcore_barrier()
        pltpu.sync_copy(acc_sh.at[pl.ds(r0, nrt)],
                        acc_out.at[cid, pl.ds(r0, nrt)])

    mesh = plsc.VectorSubcoreMesh(core_axis_name="c", subcore_axis_name="s")
    f = pl.kernel(
        body,
        out_type=jax.ShapeDtypeStruct((2, nrows, D), jnp.float32),
        mesh=mesh,
        scratch_types=[
            pltpu.VMEM((CPT, B), jnp.int32),
            pltpu.VMEM((B, D), jnp.float32),
            pltpu.VMEM_SHARED((nrows, D), jnp.float32),
        ],
    )
    return f


def _sc_scatter1(p3, idx2, zsrc):
    return _make_scatter(NP)(p3, idx2, zsrc)


def _sc_scatter2(p3, idx2, zsrc):
    return _make_scatter(NG)(p3, idx2, zsrc)


def _post_body(a0_ref, a1_ref, e0_ref, e1_ref, res_ref, wo0, wo1t, bo,
               gl, bl, o_ref):
    sv = a0_ref[...] + a1_ref[...]
    e = e0_ref[...] + e1_ref[...]
    ssum = e[:, :H]
    st = e[:, H:]
    d = ssum + 1e-16
    rep = _rep_sel()
    aves = sv / _dot(d, rep)
    avet = st / d
    ssp = _head_sum_sel()
    c2 = _dot(aves * aves, ssp) - avet * avet
    denom = jnp.sqrt(jnp.clip(jnp.abs(c2), 1e-8, None))
    outs = aves * _dot(1.0 / denom, rep)
    tt = jnp.sqrt(jnp.sum(outs * outs, axis=-1, keepdims=True) + 1.0)
    s2 = tt * wo0[...] + _dot(outs, wo1t[...]) + bo[...]
    t2 = jnp.sqrt(jnp.sum(s2 * s2, axis=-1, keepdims=True) + 1.0)
    rs = res_ref[...]
    rt = jnp.sqrt(jnp.sum(rs * rs, axis=-1, keepdims=True) + 1.0)
    a2s = 0.5 * (s2 + rs)
    a2t = 0.5 * (t2 + rt)
    c3 = jnp.sum(a2s * a2s, axis=-1, keepdims=True) - a2t * a2t
    den3 = jnp.sqrt(jnp.clip(jnp.abs(c3), 1e-8, None))
    hs3 = a2s / den3
    m = jnp.mean(hs3, axis=-1, keepdims=True)
    v = jnp.mean((hs3 - m) ** 2, axis=-1, keepdims=True)
    o_ref[...] = (hs3 - m) / jnp.sqrt(v + 1e-5) * gl[...] + bl[...]


def _tc_post(a0, a1, e0, e1, res, wo0, wo1t, bo, gl, bl):
    row = pl.BlockSpec((RB, D), lambda i: (i, 0))
    erow = pl.BlockSpec((RB, 16), lambda i: (i, 0))
    wfull = pl.BlockSpec((1, D), lambda i: (0, 0))
    wmat = pl.BlockSpec((D, D), lambda i: (0, 0))
    return pl.pallas_call(
        _post_body,
        grid=(NP // RB,),
        in_specs=[row, row, erow, erow, row, wfull, wmat, wfull, wfull,
                  wfull],
        out_specs=row,
        out_shape=jax.ShapeDtypeStruct((NP, D), jnp.float32),
    )(a0, a1, e0, e1, res, wo0, wo1t, bo, gl, bl)


def _final_body(h_ref, w0, w1t, b, o_ref):
    hs = h_ref[...]
    t = jnp.sqrt(jnp.sum(hs * hs, axis=-1, keepdims=True) + 1.0)
    s = t * w0[...] + _dot(hs, w1t[...]) + b[...]
    st2 = jnp.sum(s * s, axis=-1, keepdims=True)
    xt = jnp.clip(jnp.sqrt(st2 + 1.0), 1.0 + 1e-7, None)
    dd = jnp.log(xt + jnp.sqrt(xt * xt - 1.0))
    nrm = jnp.sqrt(jnp.clip(st2, 1e-12, None))
    o_ref[...] = dd * s / nrm


def _tc_final(hs, w0, w1t, b):
    row = pl.BlockSpec((RB, D), lambda i: (i, 0))
    wfull = pl.BlockSpec((1, D), lambda i: (0, 0))
    wmat = pl.BlockSpec((D, D), lambda i: (0, 0))
    return pl.pallas_call(
        _final_body,
        grid=(NP // RB,),
        in_specs=[row, wfull, wmat, wfull],
        out_specs=row,
        out_shape=jax.ShapeDtypeStruct((NP, D), jnp.float32),
    )(hs, w0, w1t, b)


def kernel(x, edge_index, edge_feats, W_in, b_in, g0, be0, Wq, bq, Wk, bk,
           Wv, bv, Wo, bo, We, gl, bl, W_out, b_out):
    xp = jnp.pad(x, ((0, NP - N), (0, 0)))
    src = edge_index[0]
    dst = edge_index[1]
    srcp = jnp.concatenate([src, jnp.zeros((EP - E,), jnp.int32)])
    dstp = jnp.concatenate([dst, jnp.full((EP - E,), NP - 1, jnp.int32)])
    src2 = srcp.reshape(CHUNKS, B)
    dst2 = dstp.reshape(CHUNKS, B)
    dstg2 = (dstp // 8).reshape(CHUNKS, B)
    mask8 = jax.nn.one_hot(dstp % 8, 8, dtype=jnp.float32)
    efp = jnp.pad(edge_feats, ((0, EP - E), (0, 0)))
    zsrc = jnp.zeros((NPT, D), jnp.float32)

    hs = _tc_pre(xp, W_in[:, 0][None], W_in[:, 1:].T, b_in[None],
                 g0[None], be0[None])
    for l in range(L):
        wq = Wq[l].reshape(D, D + 1)
        wk = Wk[l].reshape(D, D + 1)
        wv = Wv[l].reshape(D, D + 1)
        qtab, ktab, vtab = _tc_tabs(
            hs,
            wq[:, 0][None], wq[:, 1:].T, bq[l].reshape(1, D),
            wk[:, 0][None], wk[:, 1:].T, bk[l].reshape(1, D),
            wv[:, 0][None], wv[:, 1:].T, bv[l].reshape(1, D))
        qe, ke, ve = _sc_gather(qtab, ktab, vtab, src2, dst2)
        p1, p2 = _tc_score(qe.reshape(EP, D), ke.reshape(EP, D),
                           ve.reshape(EP, D), efp, mask8, We[l].T)
        acc1 = _sc_scatter1(p1.reshape(CHUNKS, B, D), dst2, zsrc)
        acc2 = _sc_scatter2(p2.reshape(CHUNKS, B, D), dstg2, zsrc)
        acc2u = acc2.reshape(2, NP, 16)
        wo = Wo[l]
        hs = _tc_post(acc1[0], acc1[1], acc2u[0], acc2u[1], hs,
                      wo[:, 0][None], wo[:, 1:].T, bo[l][None],
                      gl[l][None], bl[l][None])
    out = _tc_final(hs, W_out[:, 0][None], W_out[:, 1:].T, b_out[None])
    return out[:N]


# speedup vs baseline: 28.2399x; 1.2067x over previous
import jax
import jax.numpy as jnp
from jax import lax
from jax.experimental import pallas as pl
from jax.experimental.pallas import tpu as pltpu
from jax.experimental.pallas import tpu_sc as plsc

N = 10000
E = 320000
D = 128
H = 8
HD = 16
L = 2
EDIM = 16
NP = 10240
EP = 327680
B = 128
CHUNKS = EP // B
NW = 32
CPT = CHUNKS // NW
NG = NP // 8
RB = 1024
EB = 2048
NPT = NP // 16
NGT = NG // 16


def _f32(x):
    return x.astype(jnp.float32)


def _dot(a, b):
    return jnp.dot(a, b, preferred_element_type=jnp.float32)


def _head_sum_sel():
    c = lax.broadcasted_iota(jnp.int32, (D, H), 0)
    h = lax.broadcasted_iota(jnp.int32, (D, H), 1)
    return _f32(c // HD == h)


def _rep_sel():
    h = lax.broadcasted_iota(jnp.int32, (H, D), 0)
    c = lax.broadcasted_iota(jnp.int32, (H, D), 1)
    return _f32(c // HD == h)


def _grp_sel():
    g = lax.broadcasted_iota(jnp.int32, (8, D), 0)
    c = lax.broadcasted_iota(jnp.int32, (8, D), 1)
    return _f32(c // 16 == g)


def _tile16_sel():
    j = lax.broadcasted_iota(jnp.int32, (16, D), 0)
    c = lax.broadcasted_iota(jnp.int32, (16, D), 1)
    return _f32(c % 16 == j)


def _pre_body(x_ref, w0_ref, w1t_ref, b_ref, g_ref, be_ref, o_ref):
    x = x_ref[...]
    nrm = jnp.sqrt(jnp.clip(jnp.sum(x * x, axis=-1, keepdims=True), 1e-12, None))
    en = jnp.exp(nrm)
    inv_en = 1.0 / en
    zt = 0.5 * (en + inv_en)
    zs = x * (0.5 * (en - inv_en) / nrm)
    s1 = zt * w0_ref[...] + _dot(zs, w1t_ref[...]) + b_ref[...]
    m = jnp.mean(s1, axis=-1, keepdims=True)
    v = jnp.mean((s1 - m) ** 2, axis=-1, keepdims=True)
    s2 = (s1 - m) / jnp.sqrt(v + 1e-5) * g_ref[...] + be_ref[...]
    o_ref[...] = jnp.maximum(s2, 0.0)


def _tc_pre(xp, w0, w1t, b, g, be):
    row = pl.BlockSpec((RB, D), lambda i: (i, 0))
    wfull = pl.BlockSpec((1, D), lambda i: (0, 0))
    wmat = pl.BlockSpec((D, D), lambda i: (0, 0))
    return pl.pallas_call(
        _pre_body,
        grid=(NP // RB,),
        in_specs=[row, wfull, wmat, wfull, wfull, wfull],
        out_specs=row,
        out_shape=jax.ShapeDtypeStruct((NP, D), jnp.float32),
    )(xp, w0, w1t, b, g, be)


def _tabs_body(h_ref, q0, q1t, qb, k0, k1t, kb, v0, v1t, vb,
               qo_ref, ko_ref, vo_ref):
    hs = h_ref[...]
    t = jnp.sqrt(jnp.sum(hs * hs, axis=-1, keepdims=True) + 1.0)
    qo_ref[...] = t * q0[...] + _dot(hs, q1t[...]) + qb[...]
    ko_ref[...] = 0.5 * (t * k0[...] + _dot(hs, k1t[...]) + kb[...])
    vo_ref[...] = t * v0[...] + _dot(hs, v1t[...]) + vb[...]


def _tc_tabs(hs, q0, q1t, qb, k0, k1t, kb, v0, v1t, vb):
    row = pl.BlockSpec((RB, D), lambda i: (i, 0))
    wfull = pl.BlockSpec((1, D), lambda i: (0, 0))
    wmat = pl.BlockSpec((D, D), lambda i: (0, 0))
    osh = jax.ShapeDtypeStruct((NP, D), jnp.float32)
    return pl.pallas_call(
        _tabs_body,
        grid=(NP // RB,),
        in_specs=[row] + [wfull, wmat, wfull] * 3,
        out_specs=[row, row, row],
        out_shape=[osh, osh, osh],
    )(hs, q0, q1t, qb, k0, k1t, kb, v0, v1t, vb)


def _gather_body(qtab, ktab, vtab, src2, dst2, qe, ke, ve,
                 idxs, idxd,
                 qb0, kb0, vb0, qb1, kb1, vb1,
                 sq0, sk0, sv0, sq1, sk1, sv1):
    cid = lax.axis_index("c")
    sid = lax.axis_index("s")
    w = sid * 2 + cid
    base = w * CPT
    pltpu.sync_copy(dst2.at[pl.ds(base, CPT)], idxd)
    pltpu.sync_copy(src2.at[pl.ds(base, CPT)], idxs)
    bufs = ((qb0, kb0, vb0, sq0, sk0, sv0),
            (qb1, kb1, vb1, sq1, sk1, sv1))

    def issue(c, bs):
        qb, kb, vb, sq, sk, sv = bs
        pltpu.async_copy(qtab.at[idxd.at[c]], qb, sq)
        pltpu.async_copy(ktab.at[idxs.at[c]], kb, sk)
        pltpu.async_copy(vtab.at[idxs.at[c]], vb, sv)

    def wait(c, bs):
        qb, kb, vb, sq, sk, sv = bs
        pltpu.make_async_copy(qtab.at[idxd.at[c]], qb, sq).wait()
        pltpu.make_async_copy(ktab.at[idxs.at[c]], kb, sk).wait()
        pltpu.make_async_copy(vtab.at[idxs.at[c]], vb, sv).wait()

    def write(c, bs):
        qb, kb, vb = bs[0], bs[1], bs[2]
        pltpu.sync_copy(qb, qe.at[base + c])
        pltpu.sync_copy(kb, ke.at[base + c])
        pltpu.sync_copy(vb, ve.at[base + c])

    issue(0, bufs[0])

    def step2(i, carry):
        c0 = 2 * i
        issue(c0 + 1, bufs[1])
        wait(c0, bufs[0])
        write(c0, bufs[0])

        @pl.when(i < CPT // 2 - 1)
        def _():
            issue(c0 + 2, bufs[0])

        wait(c0 + 1, bufs[1])
        write(c0 + 1, bufs[1])
        return carry

    lax.fori_loop(0, CPT // 2, step2, 0)


def _sc_gather(qtab, ktab, vtab, src2, dst2):
    mesh = plsc.VectorSubcoreMesh(core_axis_name="c", subcore_axis_name="s")
    osh = jax.ShapeDtypeStruct((CHUNKS, B, D), jnp.float32)
    f = pl.kernel(
        _gather_body,
        out_type=[osh, osh, osh],
        mesh=mesh,
        scratch_types=[
            pltpu.VMEM((CPT, B), jnp.int32),
            pltpu.VMEM((CPT, B), jnp.int32),
            pltpu.VMEM((B, D), jnp.float32),
            pltpu.VMEM((B, D), jnp.float32),
            pltpu.VMEM((B, D), jnp.float32),
            pltpu.VMEM((B, D), jnp.float32),
            pltpu.VMEM((B, D), jnp.float32),
            pltpu.VMEM((B, D), jnp.float32),
            pltpu.SemaphoreType.DMA,
            pltpu.SemaphoreType.DMA,
            pltpu.SemaphoreType.DMA,
            pltpu.SemaphoreType.DMA,
            pltpu.SemaphoreType.DMA,
            pltpu.SemaphoreType.DMA,
        ],
    )
    return f(qtab, ktab, vtab, src2, dst2)


def _score_body(qe_ref, ke_ref, ve_ref, ef_ref, mask_ref, wet_ref,
                p1_ref, p2_ref):
    qe = qe_ref[...]
    ke = ke_ref[...]
    ve = ve_ref[...]
    ssp = _head_sum_sel()
    tq = jnp.sqrt(_dot(qe * qe, ssp) + 1.0)
    tk = jnp.sqrt(4.0 * _dot(ke * ke, ssp) + 1.0)
    sc = _dot(qe * ke, ssp) - 0.5 * tq * tk
    efs = _dot(ef_ref[...], wet_ref[...]) + 0.5
    ex = jnp.exp(sc + efs)
    rep = _rep_sel()
    p1_ref[...] = _dot(ex, rep) * ve
    tv = jnp.sqrt(_dot(ve * ve, ssp) + 1.0)
    ext = jnp.concatenate([ex, ex * tv], axis=-1)
    mrep = _dot(mask_ref[...], _grp_sel())
    erep = _dot(ext, _tile16_sel())
    p2_ref[...] = mrep * erep


def _tc_score(qe, ke, ve, efp, mask8, wet):
    erow = pl.BlockSpec((EB, D), lambda i: (i, 0))
    efrow = pl.BlockSpec((EB, EDIM), lambda i: (i, 0))
    mrow = pl.BlockSpec((EB, 8), lambda i: (i, 0))
    wspec = pl.BlockSpec((EDIM, H), lambda i: (0, 0))
    osh = jax.ShapeDtypeStruct((EP, D), jnp.float32)
    return pl.pallas_call(
        _score_body,
        grid=(EP // EB,),
        in_specs=[erow, erow, erow, efrow, mrow, wspec],
        out_specs=[erow, erow],
        out_shape=[osh, osh],
    )(qe, ke, ve, efp, mask8, wet)


def _make_scatter(nrows):
    nrt = nrows // 16

    def body(p3, idx2, zsrc, acc_out, idxv, pb0, pb1, acc_sh, sp0, sp1):
        cid = lax.axis_index("c")
        sid = lax.axis_index("s")
        w = sid * 2 + cid
        base = w * CPT
        pltpu.sync_copy(idx2.at[pl.ds(base, CPT)], idxv)
        r0 = sid * nrt
        pltpu.sync_copy(zsrc.at[pl.ds(0, nrt)], acc_sh.at[pl.ds(r0, nrt)])
        plsc.subcore_barrier()
        pbufs = ((pb0, sp0), (pb1, sp1))
        pltpu.async_copy(p3.at[base], pb0, sp0)

        def step2(i, carry):
            c0 = 2 * i
            pltpu.async_copy(p3.at[base + c0 + 1], pb1, sp1)
            pltpu.make_async_copy(p3.at[base + c0], pb0, sp0).wait()
            pltpu.sync_copy(pb0, acc_sh.at[idxv.at[c0]], add=True)

            @pl.when(i < CPT // 2 - 1)
            def _():
                pltpu.async_copy(p3.at[base + c0 + 2], pb0, sp0)

            pltpu.make_async_copy(p3.at[base + c0 + 1], pb1, sp1).wait()
            pltpu.sync_copy(pb1, acc_sh.at[idxv.at[c0 + 1]], add=True)
            return carry

        lax.fori_loop(0, CPT // 2, step2, 0)
        plsc.subcore_barrier()
        pltpu.sync_copy(acc_sh.at[pl.ds(r0, nrt)],
                        acc_out.at[cid, pl.ds(r0, nrt)])

    mesh = plsc.VectorSubcoreMesh(core_axis_name="c", subcore_axis_name="s")
    f = pl.kernel(
        body,
        out_type=jax.ShapeDtypeStruct((2, nrows, D), jnp.float32),
        mesh=mesh,
        scratch_types=[
            pltpu.VMEM((CPT, B), jnp.int32),
            pltpu.VMEM((B, D), jnp.float32),
            pltpu.VMEM((B, D), jnp.float32),
            pltpu.VMEM_SHARED((nrows, D), jnp.float32),
            pltpu.SemaphoreType.DMA,
            pltpu.SemaphoreType.DMA,
        ],
    )
    return f


def _sc_scatter1(p3, idx2, zsrc):
    return _make_scatter(NP)(p3, idx2, zsrc)


def _sc_scatter2(p3, idx2, zsrc):
    return _make_scatter(NG)(p3, idx2, zsrc)


def _post_body(a0_ref, a1_ref, e0_ref, e1_ref, res_ref, wo0, wo1t, bo,
               gl, bl, o_ref):
    sv = a0_ref[...] + a1_ref[...]
    e = e0_ref[...] + e1_ref[...]
    ssum = e[:, :H]
    st = e[:, H:]
    d = ssum + 1e-16
    rep = _rep_sel()
    aves = sv / _dot(d, rep)
    avet = st / d
    ssp = _head_sum_sel()
    c2 = _dot(aves * aves, ssp) - avet * avet
    denom = jnp.sqrt(jnp.clip(jnp.abs(c2), 1e-8, None))
    outs = aves * _dot(1.0 / denom, rep)
    tt = jnp.sqrt(jnp.sum(outs * outs, axis=-1, keepdims=True) + 1.0)
    s2 = tt * wo0[...] + _dot(outs, wo1t[...]) + bo[...]
    t2 = jnp.sqrt(jnp.sum(s2 * s2, axis=-1, keepdims=True) + 1.0)
    rs = res_ref[...]
    rt = jnp.sqrt(jnp.sum(rs * rs, axis=-1, keepdims=True) + 1.0)
    a2s = 0.5 * (s2 + rs)
    a2t = 0.5 * (t2 + rt)
    c3 = jnp.sum(a2s * a2s, axis=-1, keepdims=True) - a2t * a2t
    den3 = jnp.sqrt(jnp.clip(jnp.abs(c3), 1e-8, None))
    hs3 = a2s / den3
    m = jnp.mean(hs3, axis=-1, keepdims=True)
    v = jnp.mean((hs3 - m) ** 2, axis=-1, keepdims=True)
    o_ref[...] = (hs3 - m) / jnp.sqrt(v + 1e-5) * gl[...] + bl[...]


def _tc_post(a0, a1, e0, e1, res, wo0, wo1t, bo, gl, bl):
    row = pl.BlockSpec((RB, D), lambda i: (i, 0))
    erow = pl.BlockSpec((RB, 16), lambda i: (i, 0))
    wfull = pl.BlockSpec((1, D), lambda i: (0, 0))
    wmat = pl.BlockSpec((D, D), lambda i: (0, 0))
    return pl.pallas_call(
        _post_body,
        grid=(NP // RB,),
        in_specs=[row, row, erow, erow, row, wfull, wmat, wfull, wfull,
                  wfull],
        out_specs=row,
        out_shape=jax.ShapeDtypeStruct((NP, D), jnp.float32),
    )(a0, a1, e0, e1, res, wo0, wo1t, bo, gl, bl)


def _final_body(h_ref, w0, w1t, b, o_ref):
    hs = h_ref[...]
    t = jnp.sqrt(jnp.sum(hs * hs, axis=-1, keepdims=True) + 1.0)
    s = t * w0[...] + _dot(hs, w1t[...]) + b[...]
    st2 = jnp.sum(s * s, axis=-1, keepdims=True)
    xt = jnp.clip(jnp.sqrt(st2 + 1.0), 1.0 + 1e-7, None)
    dd = jnp.log(xt + jnp.sqrt(xt * xt - 1.0))
    nrm = jnp.sqrt(jnp.clip(st2, 1e-12, None))
    o_ref[...] = dd * s / nrm


def _tc_final(hs, w0, w1t, b):
    row = pl.BlockSpec((RB, D), lambda i: (i, 0))
    wfull = pl.BlockSpec((1, D), lambda i: (0, 0))
    wmat = pl.BlockSpec((D, D), lambda i: (0, 0))
    return pl.pallas_call(
        _final_body,
        grid=(NP // RB,),
        in_specs=[row, wfull, wmat, wfull],
        out_specs=row,
        out_shape=jax.ShapeDtypeStruct((NP, D), jnp.float32),
    )(hs, w0, w1t, b)


def kernel(x, edge_index, edge_feats, W_in, b_in, g0, be0, Wq, bq, Wk, bk,
           Wv, bv, Wo, bo, We, gl, bl, W_out, b_out):
    xp = jnp.pad(x, ((0, NP - N), (0, 0)))
    src = edge_index[0]
    dst = edge_index[1]
    srcp = jnp.concatenate([src, jnp.zeros((EP - E,), jnp.int32)])
    dstp = jnp.concatenate([dst, jnp.full((EP - E,), NP - 1, jnp.int32)])
    src2 = srcp.reshape(CHUNKS, B)
    dst2 = dstp.reshape(CHUNKS, B)
    dstg2 = (dstp // 8).reshape(CHUNKS, B)
    mask8 = jax.nn.one_hot(dstp % 8, 8, dtype=jnp.float32)
    efp = jnp.pad(edge_feats, ((0, EP - E), (0, 0)))
    zsrc = jnp.zeros((NPT, D), jnp.float32)

    hs = _tc_pre(xp, W_in[:, 0][None], W_in[:, 1:].T, b_in[None],
                 g0[None], be0[None])
    for l in range(L):
        wq = Wq[l].reshape(D, D + 1)
        wk = Wk[l].reshape(D, D + 1)
        wv = Wv[l].reshape(D, D + 1)
        qtab, ktab, vtab = _tc_tabs(
            hs,
            wq[:, 0][None], wq[:, 1:].T, bq[l].reshape(1, D),
            wk[:, 0][None], wk[:, 1:].T, bk[l].reshape(1, D),
            wv[:, 0][None], wv[:, 1:].T, bv[l].reshape(1, D))
        qe, ke, ve = _sc_gather(qtab, ktab, vtab, src2, dst2)
        p1, p2 = _tc_score(qe.reshape(EP, D), ke.reshape(EP, D),
                           ve.reshape(EP, D), efp, mask8, We[l].T)
        acc1 = _sc_scatter1(p1.reshape(CHUNKS, B, D), dst2, zsrc)
        acc2 = _sc_scatter2(p2.reshape(CHUNKS, B, D), dstg2, zsrc)
        acc2u = acc2.reshape(2, NP, 16)
        wo = Wo[l]
        hs = _tc_post(acc1[0], acc1[1], acc2u[0], acc2u[1], hs,
                      wo[:, 0][None], wo[:, 1:].T, bo[l][None],
                      gl[l][None], bl[l][None])
    out = _tc_final(hs, W_out[:, 0][None], W_out[:, 1:].T, b_out[None])
    return out[:N]
